# baseline (speedup 1.0000x reference)
import functools

import jax
import jax.numpy as jnp
from jax import lax
from jax.experimental import pallas as pl
from jax.experimental.pallas import tpu as pltpu
from jax.experimental.pallas import tpu_sc as plsc

N = 10000
D = 128
H = 128
DEC = 64

NSC = 2
NTILE = 16
LANE = 128

NP = 10240
ROWS_PER_TILE = NP // NTILE

EP = 327680
ECH = EP // (NSC * NTILE) // LANE
MP = 8192
MCH = MP // (NSC * NTILE) // LANE

_mesh = plsc.VectorSubcoreMesh(core_axis_name="c", subcore_axis_name="s")


@functools.partial(
    pl.kernel,
    out_type=(jax.ShapeDtypeStruct((NSC, NP), jnp.float32),
              jax.ShapeDtypeStruct((NSC, NP), jnp.float32)),
    mesh=_mesh,
    scratch_types=[
        pltpu.VMEM((ECH, LANE), jnp.int32),
        pltpu.VMEM((MCH, LANE), jnp.int32),
        pltpu.VMEM((LANE,), jnp.float32),
        pltpu.VMEM_SHARED((NP,), jnp.float32),
        pltpu.VMEM_SHARED((NP,), jnp.float32),
    ],
)
def _sc_hist(ei_hbm, midx_hbm, z1_hbm, deg_out, msk_out,
             didx_v, midx_v, ones_v, deg_sh, msk_sh):
    c = lax.axis_index("c")
    s = lax.axis_index("s")
    t = c * NTILE + s
    pltpu.sync_copy(ei_hbm.at[1, pl.ds(t * ECH, ECH)], didx_v)
    pltpu.sync_copy(midx_hbm.at[pl.ds(t * MCH, MCH)], midx_v)
    for i in range(LANE // 16):
        ones_v[pl.ds(i * 16, 16)] = jnp.ones((16,), jnp.float32)
    r0 = s * ROWS_PER_TILE
    pltpu.sync_copy(z1_hbm.at[pl.ds(r0, ROWS_PER_TILE)],
                    deg_sh.at[pl.ds(r0, ROWS_PER_TILE)])
    pltpu.sync_copy(z1_hbm.at[pl.ds(r0, ROWS_PER_TILE)],
                    msk_sh.at[pl.ds(r0, ROWS_PER_TILE)])
    plsc.subcore_barrier()

    @pl.loop(0, ECH)
    def _(j):
        pltpu.sync_copy(ones_v, deg_sh.at[didx_v.at[j]], add=True)

    for j in range(MCH):
        pltpu.sync_copy(ones_v, msk_sh.at[midx_v.at[j]], add=True)
    plsc.subcore_barrier()
    pltpu.sync_copy(deg_sh.at[pl.ds(r0, ROWS_PER_TILE)],
                    deg_out.at[c, pl.ds(r0, ROWS_PER_TILE)])
    pltpu.sync_copy(msk_sh.at[pl.ds(r0, ROWS_PER_TILE)],
                    msk_out.at[c, pl.ds(r0, ROWS_PER_TILE)])


W64 = 64
NBUF = 2
NOUT = ECH // NBUF
NREM = ECH - NOUT * NBUF


@functools.partial(
    pl.kernel,
    out_type=jax.ShapeDtypeStruct((NSC, NP, W64), jnp.float32),
    mesh=_mesh,
    scratch_types=[
        pltpu.VMEM((ECH, LANE), jnp.int32),
        pltpu.VMEM((ECH, LANE), jnp.int32),
        pltpu.VMEM_SHARED((NP, W64), jnp.float32),
        pltpu.VMEM_SHARED((NP, W64), jnp.float32),
    ] + [pltpu.VMEM((LANE, W64), jnp.float32)] * NBUF
      + [pltpu.SemaphoreType.DMA] * NBUF,
    compiler_params=pltpu.CompilerParams(use_tc_tiling_on_sc=False),
)
def _sc_edge64(tbl_hbm, ei_hbm, z2_hbm, out_hbm,
               sidx_v, didx_v, tbl_sh, acc_sh, *rest):
    rows = rest[:NBUF]
    sems = rest[NBUF:]
    c = lax.axis_index("c")
    s = lax.axis_index("s")
    t = c * NTILE + s
    pltpu.sync_copy(ei_hbm.at[0, pl.ds(t * ECH, ECH)], sidx_v)
    pltpu.sync_copy(ei_hbm.at[1, pl.ds(t * ECH, ECH)], didx_v)
    r0 = s * ROWS_PER_TILE
    pltpu.sync_copy(tbl_hbm.at[pl.ds(r0, ROWS_PER_TILE)],
                    tbl_sh.at[pl.ds(r0, ROWS_PER_TILE)])
    pltpu.sync_copy(z2_hbm.at[pl.ds(r0, ROWS_PER_TILE)],
                    acc_sh.at[pl.ds(r0, ROWS_PER_TILE)])
    plsc.subcore_barrier()

    for b in range(NBUF):
        pltpu.async_copy(tbl_sh.at[sidx_v.at[b]], rows[b], sems[b])

    @pl.loop(0, NOUT)
    def _(o):
        for b in range(NBUF):
            j = o * NBUF + b
            pltpu.make_async_copy(
                tbl_sh.at[sidx_v.at[j]], rows[b], sems[b]).wait()
            pltpu.sync_copy(rows[b], acc_sh.at[didx_v.at[j]], add=True)

            @pl.when(o < NOUT - 1 + (1 if b < NREM else 0))
            def _():
                pltpu.async_copy(
                    tbl_sh.at[sidx_v.at[j + NBUF]], rows[b], sems[b])

    for b in range(NREM):
        j = NOUT * NBUF + b
        pltpu.make_async_copy(
            tbl_sh.at[sidx_v.at[j]], rows[b], sems[b]).wait()
        pltpu.sync_copy(rows[b], acc_sh.at[didx_v.at[j]], add=True)

    plsc.subcore_barrier()
    pltpu.sync_copy(acc_sh.at[pl.ds(r0, ROWS_PER_TILE)],
                    out_hbm.at[c, pl.ds(r0, ROWS_PER_TILE)])


BR = 1000
GRID = N // BR


def _tc1a_body(x_ref, w_ref, mtok_ref, o_ref, m1_ref):
    o_ref[...] = jnp.dot(x_ref[...], w_ref[...],
                         preferred_element_type=jnp.float32)

    @pl.when(pl.program_id(0) == 0)
    def _():
        m1_ref[...] = jnp.dot(mtok_ref[...], w_ref[...],
                              preferred_element_type=jnp.float32)


def _tc1a(x, W_enc, mask_token):
    return pl.pallas_call(
        _tc1a_body,
        grid=(GRID,),
        in_specs=[
            pl.BlockSpec((BR, D), lambda i: (i, 0)),
            pl.BlockSpec((D, H), lambda i: (0, 0)),
            pl.BlockSpec((1, D), lambda i: (0, 0)),
        ],
        out_specs=[pl.BlockSpec((BR, H), lambda i: (i, 0)),
                   pl.BlockSpec((1, H), lambda i: (0, 0))],
        out_shape=[jax.ShapeDtypeStruct((N, H), jnp.float32),
                   jax.ShapeDtypeStruct((1, H), jnp.float32)],
    )(x, W_enc, mask_token)


def _tc1b_body(xw_ref, m1_ref, deg_ref, msk_ref, oL_ref, oR_ref):
    deg = deg_ref[0] + deg_ref[1] + 1.0
    dinv = lax.rsqrt(jnp.maximum(deg, 1.0))
    msk = msk_ref[0] + msk_ref[1]
    p1 = jnp.where(msk > 0.5, m1_ref[...], xw_ref[...]) * dinv
    oL_ref[...] = p1[:, :DEC]
    oR_ref[...] = p1[:, DEC:]


def _tc1b(XW, M1, deg2, msk2):
    return pl.pallas_call(
        _tc1b_body,
        grid=(GRID,),
        in_specs=[
            pl.BlockSpec((BR, H), lambda i: (i, 0)),
            pl.BlockSpec((1, H), lambda i: (0, 0)),
            pl.BlockSpec((NSC, BR, 1), lambda i: (0, i, 0)),
            pl.BlockSpec((NSC, BR, 1), lambda i: (0, i, 0)),
        ],
        out_specs=[pl.BlockSpec((BR, DEC), lambda i: (i, 0)),
                   pl.BlockSpec((BR, DEC), lambda i: (i, 0))],
        out_shape=[jax.ShapeDtypeStruct((NP, DEC), jnp.float32),
                   jax.ShapeDtypeStruct((NP, DEC), jnp.float32)],
    )(XW, M1, deg2, msk2)


def _tc2a_body(s1L_ref, p1L_ref, deg_ref, b_ref, w_ref, o_ref):
    deg = deg_ref[0] + deg_ref[1] + 1.0
    dinv = lax.rsqrt(jnp.maximum(deg, 1.0))
    zL = (s1L_ref[0] + s1L_ref[1] + p1L_ref[...]) * dinv + b_ref[:, :DEC]
    o_ref[...] = jnp.dot(zL, w_ref[:DEC], preferred_element_type=jnp.float32)


def _tc2a(S1L, P1L, deg2, b_enc, W_dec):
    return pl.pallas_call(
        _tc2a_body,
        grid=(GRID,),
        in_specs=[
            pl.BlockSpec((NSC, BR, DEC), lambda i: (0, i, 0)),
            pl.BlockSpec((BR, DEC), lambda i: (i, 0)),
            pl.BlockSpec((NSC, BR, 1), lambda i: (0, i, 0)),
            pl.BlockSpec((1, H), lambda i: (0, 0)),
            pl.BlockSpec((H, DEC), lambda i: (0, 0)),
        ],
        out_specs=pl.BlockSpec((BR, DEC), lambda i: (i, 0)),
        out_shape=jax.ShapeDtypeStruct((N, DEC), jnp.float32),
    )(S1L, P1L, deg2, b_enc, W_dec)


def _tc2b_body(a_ref, s1R_ref, p1R_ref, deg_ref, b_ref, w_ref, o_ref):
    deg = deg_ref[0] + deg_ref[1] + 1.0
    dinv = lax.rsqrt(jnp.maximum(deg, 1.0))
    zR = (s1R_ref[0] + s1R_ref[1] + p1R_ref[...]) * dinv + b_ref[:, DEC:]
    p2 = a_ref[...] + jnp.dot(zR, w_ref[DEC:],
                              preferred_element_type=jnp.float32)
    o_ref[...] = p2 * dinv


def _tc2b(A, S1R, P1R, deg2, b_enc, W_dec):
    return pl.pallas_call(
        _tc2b_body,
        grid=(GRID,),
        in_specs=[
            pl.BlockSpec((BR, DEC), lambda i: (i, 0)),
            pl.BlockSpec((NSC, BR, DEC), lambda i: (0, i, 0)),
            pl.BlockSpec((BR, DEC), lambda i: (i, 0)),
            pl.BlockSpec((NSC, BR, 1), lambda i: (0, i, 0)),
            pl.BlockSpec((1, H), lambda i: (0, 0)),
            pl.BlockSpec((H, DEC), lambda i: (0, 0)),
        ],
        out_specs=pl.BlockSpec((BR, DEC), lambda i: (i, 0)),
        out_shape=jax.ShapeDtypeStruct((NP, DEC), jnp.float32),
    )(A, S1R, P1R, deg2, b_enc, W_dec)


def _tc3_body(s2_ref, p2_ref, deg_ref, b_ref, w_ref, bm_ref, o_ref):
    deg = deg_ref[0] + deg_ref[1] + 1.0
    dinv = lax.rsqrt(jnp.maximum(deg, 1.0))
    h = (s2_ref[0] + s2_ref[1] + p2_ref[...]) * dinv + b_ref[...]
    h = jnp.maximum(h, 0.0)
    o_ref[...] = (jnp.dot(h, w_ref[...], preferred_element_type=jnp.float32)
                  + bm_ref[...])


def _tc3(S2, P2, deg2, b_dec, W_mlp, b_mlp):
    return pl.pallas_call(
        _tc3_body,
        grid=(GRID,),
        in_specs=[
            pl.BlockSpec((NSC, BR, DEC), lambda i: (0, i, 0)),
            pl.BlockSpec((BR, DEC), lambda i: (i, 0)),
            pl.BlockSpec((NSC, BR, 1), lambda i: (0, i, 0)),
            pl.BlockSpec((1, DEC), lambda i: (0, 0)),
            pl.BlockSpec((DEC, D), lambda i: (0, 0)),
            pl.BlockSpec((1, D), lambda i: (0, 0)),
        ],
        out_specs=pl.BlockSpec((BR, D), lambda i: (i, 0)),
        out_shape=jax.ShapeDtypeStruct((N, D), jnp.float32),
    )(S2, P2, deg2, b_dec, W_mlp, b_mlp)


def kernel(x, edge_index, mask_indices, mask_token,
           W_enc, b_enc, W_dec, b_dec, W_mlp, b_mlp):
    E = edge_index.shape[1]
    NM = mask_indices.shape[0]
    pad_id = N

    ei_p = jnp.pad(edge_index, ((0, 0), (0, EP - E)),
                   constant_values=pad_id).reshape(2, EP // LANE, LANE)
    midx_p = jnp.pad(mask_indices.astype(jnp.int32), (0, MP - NM),
                     constant_values=pad_id).reshape(MP // LANE, LANE)

    z1 = jnp.zeros((NP,), jnp.float32)
    z64 = jnp.zeros((NP, DEC), jnp.float32)

    XW, M1 = _tc1a(x, W_enc, mask_token)
    deg2, msk2 = _sc_hist(ei_p, midx_p, z1)
    deg2 = deg2.reshape(NSC, NP, 1)
    msk2 = msk2.reshape(NSC, NP, 1)

    P1L, P1R = _tc1b(XW, M1, deg2, msk2)
    S1L = _sc_edge64(P1L, ei_p, z64)
    S1R = _sc_edge64(P1R, ei_p, z64)
    A = _tc2a(S1L, P1L, deg2, b_enc.reshape(1, H), W_dec)
    P2 = _tc2b(A, S1R, P1R, deg2, b_enc.reshape(1, H), W_dec)
    S2 = _sc_edge64(P2, ei_p, z64)
    xrec = _tc3(S2, P2, deg2, b_dec.reshape(1, DEC), W_mlp,
                b_mlp.reshape(1, D))

    return (xrec, x, mask_indices)


# speedup vs baseline: 1.0282x; 1.0282x over previous
import functools

import jax
import jax.numpy as jnp
from jax import lax
from jax.experimental import pallas as pl
from jax.experimental.pallas import tpu as pltpu
from jax.experimental.pallas import tpu_sc as plsc

N = 10000
D = 128
H = 128
DEC = 64

NSC = 2
NTILE = 16
LANE = 128

NP = 10240
ROWS_PER_TILE = NP // NTILE

EP = 327680
ECH = EP // (NSC * NTILE) // LANE
MP = 8192
MCH = MP // (NSC * NTILE) // LANE

_mesh = plsc.VectorSubcoreMesh(core_axis_name="c", subcore_axis_name="s")


@functools.partial(
    pl.kernel,
    out_type=(jax.ShapeDtypeStruct((NSC, NP), jnp.float32),
              jax.ShapeDtypeStruct((NSC, NP), jnp.float32)),
    mesh=_mesh,
    scratch_types=[
        pltpu.VMEM((ECH, LANE), jnp.int32),
        pltpu.VMEM((MCH, LANE), jnp.int32),
        pltpu.VMEM((LANE,), jnp.float32),
        pltpu.VMEM_SHARED((NP,), jnp.float32),
        pltpu.VMEM_SHARED((NP,), jnp.float32),
    ],
)
def _sc_hist(ei_hbm, midx_hbm, z1_hbm, deg_out, msk_out,
             didx_v, midx_v, ones_v, deg_sh, msk_sh):
    c = lax.axis_index("c")
    s = lax.axis_index("s")
    t = c * NTILE + s
    pltpu.sync_copy(ei_hbm.at[1, pl.ds(t * ECH, ECH)], didx_v)
    pltpu.sync_copy(midx_hbm.at[pl.ds(t * MCH, MCH)], midx_v)
    for i in range(LANE // 16):
        ones_v[pl.ds(i * 16, 16)] = jnp.ones((16,), jnp.float32)
    r0 = s * ROWS_PER_TILE
    pltpu.sync_copy(z1_hbm.at[pl.ds(r0, ROWS_PER_TILE)],
                    deg_sh.at[pl.ds(r0, ROWS_PER_TILE)])
    pltpu.sync_copy(z1_hbm.at[pl.ds(r0, ROWS_PER_TILE)],
                    msk_sh.at[pl.ds(r0, ROWS_PER_TILE)])
    plsc.subcore_barrier()

    @pl.loop(0, ECH)
    def _(j):
        pltpu.sync_copy(ones_v, deg_sh.at[didx_v.at[j]], add=True)

    for j in range(MCH):
        pltpu.sync_copy(ones_v, msk_sh.at[midx_v.at[j]], add=True)
    plsc.subcore_barrier()
    pltpu.sync_copy(deg_sh.at[pl.ds(r0, ROWS_PER_TILE)],
                    deg_out.at[c, pl.ds(r0, ROWS_PER_TILE)])
    pltpu.sync_copy(msk_sh.at[pl.ds(r0, ROWS_PER_TILE)],
                    msk_out.at[c, pl.ds(r0, ROWS_PER_TILE)])


W64 = 64
NBUF = 3
NOUT = ECH // NBUF
NREM = ECH - NOUT * NBUF


@functools.partial(
    pl.kernel,
    out_type=jax.ShapeDtypeStruct((NSC, NP, W64), jnp.float32),
    mesh=_mesh,
    scratch_types=[
        pltpu.VMEM((ECH, LANE), jnp.int32),
        pltpu.VMEM((ECH, LANE), jnp.int32),
        pltpu.VMEM_SHARED((NP, W64), jnp.float32),
        pltpu.VMEM_SHARED((NP, W64), jnp.float32),
    ] + [pltpu.VMEM((LANE, W64), jnp.float32)] * NBUF
      + [pltpu.SemaphoreType.DMA] * NBUF,
    compiler_params=pltpu.CompilerParams(use_tc_tiling_on_sc=False),
)
def _sc_edge64(tbl_hbm, ei_hbm, z2_hbm, out_hbm,
               sidx_v, didx_v, tbl_sh, acc_sh, *rest):
    rows = rest[:NBUF]
    sems = rest[NBUF:]
    c = lax.axis_index("c")
    s = lax.axis_index("s")
    t = c * NTILE + s
    r0 = s * ROWS_PER_TILE
    stage = [
        pltpu.async_copy(ei_hbm.at[0, pl.ds(t * ECH, ECH)], sidx_v, sems[0]),
        pltpu.async_copy(ei_hbm.at[1, pl.ds(t * ECH, ECH)], didx_v, sems[1]),
        pltpu.async_copy(tbl_hbm.at[pl.ds(r0, ROWS_PER_TILE)],
                         tbl_sh.at[pl.ds(r0, ROWS_PER_TILE)], sems[2]),
        pltpu.async_copy(z2_hbm.at[pl.ds(r0, ROWS_PER_TILE)],
                         acc_sh.at[pl.ds(r0, ROWS_PER_TILE)], sems[0]),
    ]
    for dsc in stage:
        dsc.wait()
    plsc.subcore_barrier()

    for b in range(NBUF):
        pltpu.async_copy(tbl_sh.at[sidx_v.at[b]], rows[b], sems[b])

    @pl.loop(0, NOUT)
    def _(o):
        for b in range(NBUF):
            j = o * NBUF + b
            pltpu.make_async_copy(
                tbl_sh.at[sidx_v.at[j]], rows[b], sems[b]).wait()
            pltpu.sync_copy(rows[b], acc_sh.at[didx_v.at[j]], add=True)

            @pl.when(o < NOUT - 1 + (1 if b < NREM else 0))
            def _():
                pltpu.async_copy(
                    tbl_sh.at[sidx_v.at[j + NBUF]], rows[b], sems[b])

    for b in range(NREM):
        j = NOUT * NBUF + b
        pltpu.make_async_copy(
            tbl_sh.at[sidx_v.at[j]], rows[b], sems[b]).wait()
        pltpu.sync_copy(rows[b], acc_sh.at[didx_v.at[j]], add=True)

    plsc.subcore_barrier()
    pltpu.sync_copy(acc_sh.at[pl.ds(r0, ROWS_PER_TILE)],
                    out_hbm.at[c, pl.ds(r0, ROWS_PER_TILE)])


BR = 1000
GRID = N // BR


def _tc1a_body(x_ref, w_ref, mtok_ref, o_ref, m1_ref):
    o_ref[...] = jnp.dot(x_ref[...], w_ref[...],
                         preferred_element_type=jnp.float32)

    @pl.when(pl.program_id(0) == 0)
    def _():
        m1_ref[...] = jnp.dot(mtok_ref[...], w_ref[...],
                              preferred_element_type=jnp.float32)


def _tc1a(x, W_enc, mask_token):
    return pl.pallas_call(
        _tc1a_body,
        grid=(GRID,),
        in_specs=[
            pl.BlockSpec((BR, D), lambda i: (i, 0)),
            pl.BlockSpec((D, H), lambda i: (0, 0)),
            pl.BlockSpec((1, D), lambda i: (0, 0)),
        ],
        out_specs=[pl.BlockSpec((BR, H), lambda i: (i, 0)),
                   pl.BlockSpec((1, H), lambda i: (0, 0))],
        out_shape=[jax.ShapeDtypeStruct((N, H), jnp.float32),
                   jax.ShapeDtypeStruct((1, H), jnp.float32)],
    )(x, W_enc, mask_token)


def _tc1b_body(xw_ref, m1_ref, deg_ref, msk_ref, oL_ref, oR_ref):
    deg = deg_ref[0] + deg_ref[1] + 1.0
    dinv = lax.rsqrt(jnp.maximum(deg, 1.0))
    msk = msk_ref[0] + msk_ref[1]
    p1 = jnp.where(msk > 0.5, m1_ref[...], xw_ref[...]) * dinv
    oL_ref[...] = p1[:, :DEC]
    oR_ref[...] = p1[:, DEC:]


def _tc1b(XW, M1, deg2, msk2):
    return pl.pallas_call(
        _tc1b_body,
        grid=(GRID,),
        in_specs=[
            pl.BlockSpec((BR, H), lambda i: (i, 0)),
            pl.BlockSpec((1, H), lambda i: (0, 0)),
            pl.BlockSpec((NSC, BR, 1), lambda i: (0, i, 0)),
            pl.BlockSpec((NSC, BR, 1), lambda i: (0, i, 0)),
        ],
        out_specs=[pl.BlockSpec((BR, DEC), lambda i: (i, 0)),
                   pl.BlockSpec((BR, DEC), lambda i: (i, 0))],
        out_shape=[jax.ShapeDtypeStruct((NP, DEC), jnp.float32),
                   jax.ShapeDtypeStruct((NP, DEC), jnp.float32)],
    )(XW, M1, deg2, msk2)


def _tc2a_body(s1L_ref, p1L_ref, deg_ref, b_ref, w_ref, o_ref):
    deg = deg_ref[0] + deg_ref[1] + 1.0
    dinv = lax.rsqrt(jnp.maximum(deg, 1.0))
    zL = (s1L_ref[0] + s1L_ref[1] + p1L_ref[...]) * dinv + b_ref[:, :DEC]
    o_ref[...] = jnp.dot(zL, w_ref[:DEC], preferred_element_type=jnp.float32)


def _tc2a(S1L, P1L, deg2, b_enc, W_dec):
    return pl.pallas_call(
        _tc2a_body,
        grid=(GRID,),
        in_specs=[
            pl.BlockSpec((NSC, BR, DEC), lambda i: (0, i, 0)),
            pl.BlockSpec((BR, DEC), lambda i: (i, 0)),
            pl.BlockSpec((NSC, BR, 1), lambda i: (0, i, 0)),
            pl.BlockSpec((1, H), lambda i: (0, 0)),
            pl.BlockSpec((H, DEC), lambda i: (0, 0)),
        ],
        out_specs=pl.BlockSpec((BR, DEC), lambda i: (i, 0)),
        out_shape=jax.ShapeDtypeStruct((N, DEC), jnp.float32),
    )(S1L, P1L, deg2, b_enc, W_dec)


def _tc2b_body(a_ref, s1R_ref, p1R_ref, deg_ref, b_ref, w_ref, o_ref):
    deg = deg_ref[0] + deg_ref[1] + 1.0
    dinv = lax.rsqrt(jnp.maximum(deg, 1.0))
    zR = (s1R_ref[0] + s1R_ref[1] + p1R_ref[...]) * dinv + b_ref[:, DEC:]
    p2 = a_ref[...] + jnp.dot(zR, w_ref[DEC:],
                              preferred_element_type=jnp.float32)
    o_ref[...] = p2 * dinv


def _tc2b(A, S1R, P1R, deg2, b_enc, W_dec):
    return pl.pallas_call(
        _tc2b_body,
        grid=(GRID,),
        in_specs=[
            pl.BlockSpec((BR, DEC), lambda i: (i, 0)),
            pl.BlockSpec((NSC, BR, DEC), lambda i: (0, i, 0)),
            pl.BlockSpec((BR, DEC), lambda i: (i, 0)),
            pl.BlockSpec((NSC, BR, 1), lambda i: (0, i, 0)),
            pl.BlockSpec((1, H), lambda i: (0, 0)),
            pl.BlockSpec((H, DEC), lambda i: (0, 0)),
        ],
        out_specs=pl.BlockSpec((BR, DEC), lambda i: (i, 0)),
        out_shape=jax.ShapeDtypeStruct((NP, DEC), jnp.float32),
    )(A, S1R, P1R, deg2, b_enc, W_dec)


def _tc3_body(s2_ref, p2_ref, deg_ref, b_ref, w_ref, bm_ref, o_ref):
    deg = deg_ref[0] + deg_ref[1] + 1.0
    dinv = lax.rsqrt(jnp.maximum(deg, 1.0))
    h = (s2_ref[0] + s2_ref[1] + p2_ref[...]) * dinv + b_ref[...]
    h = jnp.maximum(h, 0.0)
    o_ref[...] = (jnp.dot(h, w_ref[...], preferred_element_type=jnp.float32)
                  + bm_ref[...])


def _tc3(S2, P2, deg2, b_dec, W_mlp, b_mlp):
    return pl.pallas_call(
        _tc3_body,
        grid=(GRID,),
        in_specs=[
            pl.BlockSpec((NSC, BR, DEC), lambda i: (0, i, 0)),
            pl.BlockSpec((BR, DEC), lambda i: (i, 0)),
            pl.BlockSpec((NSC, BR, 1), lambda i: (0, i, 0)),
            pl.BlockSpec((1, DEC), lambda i: (0, 0)),
            pl.BlockSpec((DEC, D), lambda i: (0, 0)),
            pl.BlockSpec((1, D), lambda i: (0, 0)),
        ],
        out_specs=pl.BlockSpec((BR, D), lambda i: (i, 0)),
        out_shape=jax.ShapeDtypeStruct((N, D), jnp.float32),
    )(S2, P2, deg2, b_dec, W_mlp, b_mlp)


def kernel(x, edge_index, mask_indices, mask_token,
           W_enc, b_enc, W_dec, b_dec, W_mlp, b_mlp):
    E = edge_index.shape[1]
    NM = mask_indices.shape[0]
    pad_id = N

    ei_p = jnp.pad(edge_index, ((0, 0), (0, EP - E)),
                   constant_values=pad_id).reshape(2, EP // LANE, LANE)
    midx_p = jnp.pad(mask_indices.astype(jnp.int32), (0, MP - NM),
                     constant_values=pad_id).reshape(MP // LANE, LANE)

    z1 = jnp.zeros((NP,), jnp.float32)
    z64 = jnp.zeros((NP, DEC), jnp.float32)

    XW, M1 = _tc1a(x, W_enc, mask_token)
    deg2, msk2 = _sc_hist(ei_p, midx_p, z1)
    deg2 = deg2.reshape(NSC, NP, 1)
    msk2 = msk2.reshape(NSC, NP, 1)

    P1L, P1R = _tc1b(XW, M1, deg2, msk2)
    S1L = _sc_edge64(P1L, ei_p, z64)
    S1R = _sc_edge64(P1R, ei_p, z64)
    A = _tc2a(S1L, P1L, deg2, b_enc.reshape(1, H), W_dec)
    P2 = _tc2b(A, S1R, P1R, deg2, b_enc.reshape(1, H), W_dec)
    S2 = _sc_edge64(P2, ei_p, z64)
    xrec = _tc3(S2, P2, deg2, b_dec.reshape(1, DEC), W_mlp,
                b_mlp.reshape(1, D))

    return (xrec, x, mask_indices)


# speedup vs baseline: 1.0325x; 1.0041x over previous
import functools

import jax
import jax.numpy as jnp
from jax import lax
from jax.experimental import pallas as pl
from jax.experimental.pallas import tpu as pltpu
from jax.experimental.pallas import tpu_sc as plsc

N = 10000
D = 128
H = 128
DEC = 64

NSC = 2
NTILE = 16
LANE = 128

NP = 10240
ROWS_PER_TILE = NP // NTILE

EP = 327680
ECH = EP // (NSC * NTILE) // LANE
MP = 8192
MCH = MP // (NSC * NTILE) // LANE

_mesh = plsc.VectorSubcoreMesh(core_axis_name="c", subcore_axis_name="s")


@functools.partial(
    pl.kernel,
    out_type=(jax.ShapeDtypeStruct((NSC, NP), jnp.float32),
              jax.ShapeDtypeStruct((NSC, NP), jnp.float32)),
    mesh=_mesh,
    scratch_types=[
        pltpu.VMEM((ECH, LANE), jnp.int32),
        pltpu.VMEM((MCH, LANE), jnp.int32),
        pltpu.VMEM((LANE,), jnp.float32),
        pltpu.VMEM_SHARED((NP,), jnp.float32),
        pltpu.VMEM_SHARED((NP,), jnp.float32),
        pltpu.SemaphoreType.DMA,
        pltpu.SemaphoreType.DMA,
    ],
)
def _sc_hist(ei_hbm, midx_hbm, z1_hbm, deg_out, msk_out,
             didx_v, midx_v, ones_v, deg_sh, msk_sh, sem0, sem1):
    c = lax.axis_index("c")
    s = lax.axis_index("s")
    t = c * NTILE + s
    r0 = s * ROWS_PER_TILE
    stage = [
        pltpu.async_copy(ei_hbm.at[1, pl.ds(t * ECH, ECH)], didx_v, sem0),
        pltpu.async_copy(midx_hbm.at[pl.ds(t * MCH, MCH)], midx_v, sem1),
        pltpu.async_copy(z1_hbm.at[pl.ds(r0, ROWS_PER_TILE)],
                         deg_sh.at[pl.ds(r0, ROWS_PER_TILE)], sem0),
        pltpu.async_copy(z1_hbm.at[pl.ds(r0, ROWS_PER_TILE)],
                         msk_sh.at[pl.ds(r0, ROWS_PER_TILE)], sem1),
    ]
    for i in range(LANE // 16):
        ones_v[pl.ds(i * 16, 16)] = jnp.ones((16,), jnp.float32)
    for dsc in stage:
        dsc.wait()
    plsc.subcore_barrier()

    @pl.loop(0, ECH)
    def _(j):
        pltpu.sync_copy(ones_v, deg_sh.at[didx_v.at[j]], add=True)

    for j in range(MCH):
        pltpu.sync_copy(ones_v, msk_sh.at[midx_v.at[j]], add=True)
    plsc.subcore_barrier()
    pltpu.sync_copy(deg_sh.at[pl.ds(r0, ROWS_PER_TILE)],
                    deg_out.at[c, pl.ds(r0, ROWS_PER_TILE)])
    pltpu.sync_copy(msk_sh.at[pl.ds(r0, ROWS_PER_TILE)],
                    msk_out.at[c, pl.ds(r0, ROWS_PER_TILE)])


W64 = 64
NBUF = 3
NOUT = ECH // NBUF
NREM = ECH - NOUT * NBUF


@functools.partial(
    pl.kernel,
    out_type=jax.ShapeDtypeStruct((NSC, NP, W64), jnp.float32),
    mesh=_mesh,
    scratch_types=[
        pltpu.VMEM((ECH, LANE), jnp.int32),
        pltpu.VMEM((ECH, LANE), jnp.int32),
        pltpu.VMEM_SHARED((NP, W64), jnp.float32),
        pltpu.VMEM_SHARED((NP, W64), jnp.float32),
    ] + [pltpu.VMEM((LANE, W64), jnp.float32)] * NBUF
      + [pltpu.SemaphoreType.DMA] * NBUF,
    compiler_params=pltpu.CompilerParams(use_tc_tiling_on_sc=False),
)
def _sc_edge64(tbl_hbm, ei_hbm, z2_hbm, out_hbm,
               sidx_v, didx_v, tbl_sh, acc_sh, *rest):
    rows = rest[:NBUF]
    sems = rest[NBUF:]
    c = lax.axis_index("c")
    s = lax.axis_index("s")
    t = c * NTILE + s
    r0 = s * ROWS_PER_TILE
    stage = [
        pltpu.async_copy(ei_hbm.at[0, pl.ds(t * ECH, ECH)], sidx_v, sems[0]),
        pltpu.async_copy(ei_hbm.at[1, pl.ds(t * ECH, ECH)], didx_v, sems[1]),
        pltpu.async_copy(tbl_hbm.at[pl.ds(r0, ROWS_PER_TILE)],
                         tbl_sh.at[pl.ds(r0, ROWS_PER_TILE)], sems[2]),
        pltpu.async_copy(z2_hbm.at[pl.ds(r0, ROWS_PER_TILE)],
                         acc_sh.at[pl.ds(r0, ROWS_PER_TILE)], sems[0]),
    ]
    for dsc in stage:
        dsc.wait()
    plsc.subcore_barrier()

    for b in range(NBUF):
        pltpu.async_copy(tbl_sh.at[sidx_v.at[b]], rows[b], sems[b])

    @pl.loop(0, NOUT)
    def _(o):
        for b in range(NBUF):
            j = o * NBUF + b
            pltpu.make_async_copy(
                tbl_sh.at[sidx_v.at[j]], rows[b], sems[b]).wait()
            pltpu.sync_copy(rows[b], acc_sh.at[didx_v.at[j]], add=True)

            @pl.when(o < NOUT - 1 + (1 if b < NREM else 0))
            def _():
                pltpu.async_copy(
                    tbl_sh.at[sidx_v.at[j + NBUF]], rows[b], sems[b])

    for b in range(NREM):
        j = NOUT * NBUF + b
        pltpu.make_async_copy(
            tbl_sh.at[sidx_v.at[j]], rows[b], sems[b]).wait()
        pltpu.sync_copy(rows[b], acc_sh.at[didx_v.at[j]], add=True)

    plsc.subcore_barrier()
    pltpu.sync_copy(acc_sh.at[pl.ds(r0, ROWS_PER_TILE)],
                    out_hbm.at[c, pl.ds(r0, ROWS_PER_TILE)])


BR = 1000
GRID = N // BR


def _tc1a_body(x_ref, w_ref, mtok_ref, o_ref, m1_ref):
    o_ref[...] = jnp.dot(x_ref[...], w_ref[...],
                         preferred_element_type=jnp.float32)

    @pl.when(pl.program_id(0) == 0)
    def _():
        m1_ref[...] = jnp.dot(mtok_ref[...], w_ref[...],
                              preferred_element_type=jnp.float32)


def _tc1a(x, W_enc, mask_token):
    return pl.pallas_call(
        _tc1a_body,
        grid=(GRID,),
        in_specs=[
            pl.BlockSpec((BR, D), lambda i: (i, 0)),
            pl.BlockSpec((D, H), lambda i: (0, 0)),
            pl.BlockSpec((1, D), lambda i: (0, 0)),
        ],
        out_specs=[pl.BlockSpec((BR, H), lambda i: (i, 0)),
                   pl.BlockSpec((1, H), lambda i: (0, 0))],
        out_shape=[jax.ShapeDtypeStruct((N, H), jnp.float32),
                   jax.ShapeDtypeStruct((1, H), jnp.float32)],
    )(x, W_enc, mask_token)


def _tc1b_body(xw_ref, m1_ref, deg_ref, msk_ref, oL_ref, oR_ref):
    deg = deg_ref[0] + deg_ref[1] + 1.0
    dinv = lax.rsqrt(jnp.maximum(deg, 1.0))
    msk = msk_ref[0] + msk_ref[1]
    p1 = jnp.where(msk > 0.5, m1_ref[...], xw_ref[...]) * dinv
    oL_ref[...] = p1[:, :DEC]
    oR_ref[...] = p1[:, DEC:]


def _tc1b(XW, M1, deg2, msk2):
    return pl.pallas_call(
        _tc1b_body,
        grid=(GRID,),
        in_specs=[
            pl.BlockSpec((BR, H), lambda i: (i, 0)),
            pl.BlockSpec((1, H), lambda i: (0, 0)),
            pl.BlockSpec((NSC, BR, 1), lambda i: (0, i, 0)),
            pl.BlockSpec((NSC, BR, 1), lambda i: (0, i, 0)),
        ],
        out_specs=[pl.BlockSpec((BR, DEC), lambda i: (i, 0)),
                   pl.BlockSpec((BR, DEC), lambda i: (i, 0))],
        out_shape=[jax.ShapeDtypeStruct((NP, DEC), jnp.float32),
                   jax.ShapeDtypeStruct((NP, DEC), jnp.float32)],
    )(XW, M1, deg2, msk2)


def _tc2a_body(s1L_ref, p1L_ref, deg_ref, b_ref, w_ref, o_ref):
    deg = deg_ref[0] + deg_ref[1] + 1.0
    dinv = lax.rsqrt(jnp.maximum(deg, 1.0))
    zL = (s1L_ref[0] + s1L_ref[1] + p1L_ref[...]) * dinv + b_ref[:, :DEC]
    o_ref[...] = jnp.dot(zL, w_ref[:DEC], preferred_element_type=jnp.float32)


def _tc2a(S1L, P1L, deg2, b_enc, W_dec):
    return pl.pallas_call(
        _tc2a_body,
        grid=(GRID,),
        in_specs=[
            pl.BlockSpec((NSC, BR, DEC), lambda i: (0, i, 0)),
            pl.BlockSpec((BR, DEC), lambda i: (i, 0)),
            pl.BlockSpec((NSC, BR, 1), lambda i: (0, i, 0)),
            pl.BlockSpec((1, H), lambda i: (0, 0)),
            pl.BlockSpec((H, DEC), lambda i: (0, 0)),
        ],
        out_specs=pl.BlockSpec((BR, DEC), lambda i: (i, 0)),
        out_shape=jax.ShapeDtypeStruct((N, DEC), jnp.float32),
    )(S1L, P1L, deg2, b_enc, W_dec)


def _tc2b_body(a_ref, s1R_ref, p1R_ref, deg_ref, b_ref, w_ref, o_ref):
    deg = deg_ref[0] + deg_ref[1] + 1.0
    dinv = lax.rsqrt(jnp.maximum(deg, 1.0))
    zR = (s1R_ref[0] + s1R_ref[1] + p1R_ref[...]) * dinv + b_ref[:, DEC:]
    p2 = a_ref[...] + jnp.dot(zR, w_ref[DEC:],
                              preferred_element_type=jnp.float32)
    o_ref[...] = p2 * dinv


def _tc2b(A, S1R, P1R, deg2, b_enc, W_dec):
    return pl.pallas_call(
        _tc2b_body,
        grid=(GRID,),
        in_specs=[
            pl.BlockSpec((BR, DEC), lambda i: (i, 0)),
            pl.BlockSpec((NSC, BR, DEC), lambda i: (0, i, 0)),
            pl.BlockSpec((BR, DEC), lambda i: (i, 0)),
            pl.BlockSpec((NSC, BR, 1), lambda i: (0, i, 0)),
            pl.BlockSpec((1, H), lambda i: (0, 0)),
            pl.BlockSpec((H, DEC), lambda i: (0, 0)),
        ],
        out_specs=pl.BlockSpec((BR, DEC), lambda i: (i, 0)),
        out_shape=jax.ShapeDtypeStruct((NP, DEC), jnp.float32),
    )(A, S1R, P1R, deg2, b_enc, W_dec)


def _tc3_body(s2_ref, p2_ref, deg_ref, b_ref, w_ref, bm_ref, o_ref):
    deg = deg_ref[0] + deg_ref[1] + 1.0
    dinv = lax.rsqrt(jnp.maximum(deg, 1.0))
    h = (s2_ref[0] + s2_ref[1] + p2_ref[...]) * dinv + b_ref[...]
    h = jnp.maximum(h, 0.0)
    o_ref[...] = (jnp.dot(h, w_ref[...], preferred_element_type=jnp.float32)
                  + bm_ref[...])


def _tc3(S2, P2, deg2, b_dec, W_mlp, b_mlp):
    return pl.pallas_call(
        _tc3_body,
        grid=(GRID,),
        in_specs=[
            pl.BlockSpec((NSC, BR, DEC), lambda i: (0, i, 0)),
            pl.BlockSpec((BR, DEC), lambda i: (i, 0)),
            pl.BlockSpec((NSC, BR, 1), lambda i: (0, i, 0)),
            pl.BlockSpec((1, DEC), lambda i: (0, 0)),
            pl.BlockSpec((DEC, D), lambda i: (0, 0)),
            pl.BlockSpec((1, D), lambda i: (0, 0)),
        ],
        out_specs=pl.BlockSpec((BR, D), lambda i: (i, 0)),
        out_shape=jax.ShapeDtypeStruct((N, D), jnp.float32),
    )(S2, P2, deg2, b_dec, W_mlp, b_mlp)


def kernel(x, edge_index, mask_indices, mask_token,
           W_enc, b_enc, W_dec, b_dec, W_mlp, b_mlp):
    E = edge_index.shape[1]
    NM = mask_indices.shape[0]
    pad_id = N

    ei_p = jnp.pad(edge_index, ((0, 0), (0, EP - E)),
                   constant_values=pad_id).reshape(2, EP // LANE, LANE)
    midx_p = jnp.pad(mask_indices.astype(jnp.int32), (0, MP - NM),
                     constant_values=pad_id).reshape(MP // LANE, LANE)

    z1 = jnp.zeros((NP,), jnp.float32)
    z64 = jnp.zeros((NP, DEC), jnp.float32)

    XW, M1 = _tc1a(x, W_enc, mask_token)
    deg2, msk2 = _sc_hist(ei_p, midx_p, z1)
    deg2 = deg2.reshape(NSC, NP, 1)
    msk2 = msk2.reshape(NSC, NP, 1)

    P1L, P1R = _tc1b(XW, M1, deg2, msk2)
    S1L = _sc_edge64(P1L, ei_p, z64)
    S1R = _sc_edge64(P1R, ei_p, z64)
    A = _tc2a(S1L, P1L, deg2, b_enc.reshape(1, H), W_dec)
    P2 = _tc2b(A, S1R, P1R, deg2, b_enc.reshape(1, H), W_dec)
    S2 = _sc_edge64(P2, ei_p, z64)
    xrec = _tc3(S2, P2, deg2, b_dec.reshape(1, DEC), W_mlp,
                b_mlp.reshape(1, D))

    return (xrec, x, mask_indices)
